# baseline (speedup 1.0000x reference)
import functools

import jax
import jax.numpy as jnp
from jax import lax
from jax.experimental import pallas as pl
from jax.experimental.pallas import tpu as pltpu
from jax.experimental.pallas import tpu_sc as plsc

D = 64
SCALE = 8.0
LANES = 16
IDXROW = 128


def kernel(x, lut):
    b0, hist = x.shape
    b = b0 * hist
    nw = 32
    b_per_w = b // nw
    chunk = 512
    k = chunk // IDXROW
    nchunks = b_per_w // chunk
    x2d = x.reshape(b // IDXROW, IDXROW).astype(jnp.int32)

    mesh = plsc.VectorSubcoreMesh(core_axis_name="c", subcore_axis_name="s")

    @functools.partial(
        pl.kernel,
        mesh=mesh,
        compiler_params=pltpu.CompilerParams(use_tc_tiling_on_sc=False),
        out_type=jax.ShapeDtypeStruct((b, D), jnp.float32),
        scratch_types=[
            pltpu.VMEM((k, IDXROW), jnp.int32),
            pltpu.VMEM((chunk, D), jnp.float32),
            pltpu.SemaphoreType.DMA,
        ],
    )
    def emb(idx_hbm, tab_hbm, out_hbm, idx_v, rows_v, sem):
        wid = lax.axis_index("s") * 2 + lax.axis_index("c")
        irow0 = wid * (b_per_w // IDXROW)
        obase0 = wid * b_per_w

        @pl.loop(0, nchunks)
        def chunk_body(g):
            pltpu.sync_copy(idx_hbm.at[pl.ds(irow0 + g * k, k)], idx_v)
            copies = [
                pltpu.async_copy(
                    tab_hbm.at[idx_v.at[j]],
                    rows_v.at[pl.ds(j * IDXROW, IDXROW)],
                    sem,
                )
                for j in range(k)
            ]
            for c in copies:
                c.wait()

            @pl.loop(0, chunk)
            def scale_row(i):
                for j in range(D // LANES):
                    sl = pl.ds(j * LANES, LANES)
                    rows_v[i, sl] = rows_v[i, sl] * SCALE

            pltpu.sync_copy(rows_v, out_hbm.at[pl.ds(obase0 + g * chunk, chunk)])

    out = emb(x2d, lut)
    return out.reshape(b0, hist, D)


# speedup vs baseline: 1.1260x; 1.1260x over previous
import functools

import jax
import jax.numpy as jnp
from jax import lax
from jax.experimental import pallas as pl
from jax.experimental.pallas import tpu as pltpu
from jax.experimental.pallas import tpu_sc as plsc

D = 64
SCALE = 8.0
LANES = 16
IDXROW = 128
CHUNK = 512
NBUF = 2


def kernel(x, lut):
    b0, hist = x.shape
    b = b0 * hist
    nw = 32
    b_per_w = b // nw
    k = CHUNK // IDXROW
    nchunks = b_per_w // CHUNK
    idxrows_w = b_per_w // IDXROW
    x2d = x.reshape(b // IDXROW, IDXROW).astype(jnp.int32)

    mesh = plsc.VectorSubcoreMesh(core_axis_name="c", subcore_axis_name="s")

    @functools.partial(
        pl.kernel,
        mesh=mesh,
        compiler_params=pltpu.CompilerParams(use_tc_tiling_on_sc=False),
        out_type=jax.ShapeDtypeStruct((b, D), jnp.float32),
        scratch_types=[
            pltpu.VMEM((idxrows_w, IDXROW), jnp.int32),
            [pltpu.VMEM((CHUNK, D), jnp.float32) for _ in range(NBUF)],
            [pltpu.SemaphoreType.DMA for _ in range(NBUF)],
            [pltpu.SemaphoreType.DMA for _ in range(NBUF)],
        ],
    )
    def emb(idx_hbm, tab_hbm, out_hbm, idx_v, rows_v, gsem, osem):
        wid = lax.axis_index("s") * 2 + lax.axis_index("c")
        obase0 = wid * b_per_w

        pltpu.sync_copy(idx_hbm.at[pl.ds(wid * idxrows_w, idxrows_w)], idx_v)

        def start_gathers(g, buf):
            for j in range(k):
                pltpu.async_copy(
                    tab_hbm.at[idx_v.at[g * k + j]],
                    rows_v[buf].at[pl.ds(j * IDXROW, IDXROW)],
                    gsem[buf],
                )

        def drain(sem, buf, src):
            pltpu.make_async_copy(src, rows_v[buf], sem).wait()

        start_gathers(0, 0)

        @pl.loop(0, nchunks, step=NBUF)
        def chunk_body(g0):
            for phase in range(NBUF):
                g = g0 + phase
                cur = phase
                nxt = (phase + 1) % NBUF

                @pl.when(g + 1 < nchunks)
                def _prefetch():
                    @pl.when(g >= 1)
                    def _wait_out():
                        drain(osem[nxt], nxt, out_hbm.at[pl.ds(obase0, CHUNK)])

                    start_gathers(g + 1, nxt)

                drain(gsem[cur], cur, out_hbm.at[pl.ds(obase0, CHUNK)])

                @pl.loop(0, CHUNK)
                def scale_row(i):
                    for j in range(D // LANES):
                        sl = pl.ds(j * LANES, LANES)
                        rows_v[cur][i, sl] = rows_v[cur][i, sl] * SCALE

                pltpu.async_copy(
                    rows_v[cur],
                    out_hbm.at[pl.ds(obase0 + g * CHUNK, CHUNK)],
                    osem[cur],
                )

        drain(osem[(nchunks - 1) % NBUF], (nchunks - 1) % NBUF,
              out_hbm.at[pl.ds(obase0, CHUNK)])

    out = emb(x2d, lut)
    return out.reshape(b0, hist, D)


# speedup vs baseline: 1.1312x; 1.0047x over previous
import functools

import jax
import jax.numpy as jnp
from jax import lax
from jax.experimental import pallas as pl
from jax.experimental.pallas import tpu as pltpu
from jax.experimental.pallas import tpu_sc as plsc

D = 64
SCALE = 8.0
LANES = 16
IDXROW = 128
CHUNK = 512
NBUF = 2


def kernel(x, lut):
    b0, hist = x.shape
    b = b0 * hist
    nw = 32
    b_per_w = b // nw
    k = CHUNK // IDXROW
    nchunks = b_per_w // CHUNK
    idxrows_w = b_per_w // IDXROW
    x2d = x.reshape(b // IDXROW, IDXROW).astype(jnp.int32)

    mesh = plsc.VectorSubcoreMesh(core_axis_name="c", subcore_axis_name="s")

    @functools.partial(
        pl.kernel,
        mesh=mesh,
        compiler_params=pltpu.CompilerParams(use_tc_tiling_on_sc=False),
        out_type=jax.ShapeDtypeStruct((b, D), jnp.float32),
        scratch_types=[
            pltpu.VMEM((idxrows_w, IDXROW), jnp.int32),
            [pltpu.VMEM((CHUNK, D), jnp.float32) for _ in range(NBUF)],
            [pltpu.SemaphoreType.DMA for _ in range(NBUF)],
            [pltpu.SemaphoreType.DMA for _ in range(NBUF)],
        ],
    )
    def emb(idx_hbm, tab_hbm, out_hbm, idx_v, rows_v, gsem, osem):
        wid = lax.axis_index("s") * 2 + lax.axis_index("c")
        obase0 = wid * b_per_w

        pltpu.sync_copy(idx_hbm.at[pl.ds(wid * idxrows_w, idxrows_w)], idx_v)

        def start_gathers(g, buf):
            for j in range(k):
                pltpu.async_copy(
                    tab_hbm.at[idx_v.at[g * k + j]],
                    rows_v[buf].at[pl.ds(j * IDXROW, IDXROW)],
                    gsem[buf],
                )

        def drain(sem, buf, src):
            pltpu.make_async_copy(src, rows_v[buf], sem).wait()

        start_gathers(0, 0)

        @pl.loop(0, nchunks, step=NBUF)
        def chunk_body(g0):
            for phase in range(NBUF):
                g = g0 + phase
                cur = phase
                nxt = (phase + 1) % NBUF

                @pl.when(g + 1 < nchunks)
                def _prefetch():
                    @pl.when(g >= 1)
                    def _wait_out():
                        drain(osem[nxt], nxt, out_hbm.at[pl.ds(obase0, CHUNK)])

                    start_gathers(g + 1, nxt)

                drain(gsem[cur], cur, out_hbm.at[pl.ds(obase0, CHUNK)])

                @plsc.parallel_loop(0, CHUNK, unroll=4)
                def scale_row(i):
                    for j in range(D // LANES):
                        sl = pl.ds(j * LANES, LANES)
                        rows_v[cur][i, sl] = rows_v[cur][i, sl] * SCALE

                pltpu.async_copy(
                    rows_v[cur],
                    out_hbm.at[pl.ds(obase0 + g * CHUNK, CHUNK)],
                    osem[cur],
                )

        drain(osem[(nchunks - 1) % NBUF], (nchunks - 1) % NBUF,
              out_hbm.at[pl.ds(obase0, CHUNK)])

    out = emb(x2d, lut)
    return out.reshape(b0, hist, D)
